# Initial kernel scaffold; baseline (speedup 1.0000x reference)
import jax
import jax.numpy as jnp
from jax.experimental import pallas as pl


def kernel(xyz, params):
    raise NotImplementedError("write your pallas kernel here")


# speedup vs baseline: 1.6381x; 1.6381x over previous
import functools

import jax
import jax.numpy as jnp
import numpy as np
from jax import lax
from jax.experimental import pallas as pl
from jax.experimental.pallas import tpu as pltpu

BN_EPS = 1e-5
SQ = float(np.sqrt(np.float32(1.0 + BN_EPS)))
F32 = jnp.float32
I32 = jnp.int32
HI = jax.lax.Precision.HIGHEST


def _wbs(layers):
    out = []
    for p in layers:
        out.append((p['W'], p['b'].reshape(1, -1), p['gamma'].reshape(1, -1),
                    p['beta'].reshape(1, -1)))
    return out


def _apply_layer(h, w, b, g, beta):
    h = jnp.dot(h, w, preferred_element_type=F32) + b
    h = h * g / SQ + beta
    return jnp.maximum(h, 0.0)


def _flatten(wbs):
    flat = []
    for t in wbs:
        flat.extend(t)
    return flat


def _fps_body(xt_ref, idx_ref, p2_ref, *, npoint, B, N):
    x = xt_ref[0:B, :]
    y = xt_ref[B:2 * B, :]
    z = xt_ref[2 * B:3 * B, :]
    p2_ref[...] = (x * x + y * y) + z * z
    iota = lax.broadcasted_iota(I32, (B, N), 1)
    siota = lax.broadcasted_iota(I32, (B, npoint), 1)
    biota = lax.broadcasted_iota(I32, (B, npoint), 0)

    def step(t, carry):
        dist, far, acc = carry
        acc = jnp.where(siota == t, far, acc)
        oh = iota == far
        cx = jnp.sum(jnp.where(oh, x, 0.0), axis=1, keepdims=True)
        cy = jnp.sum(jnp.where(oh, y, 0.0), axis=1, keepdims=True)
        cz = jnp.sum(jnp.where(oh, z, 0.0), axis=1, keepdims=True)
        dx = x - cx
        dy = y - cy
        dz = z - cz
        d = (dx * dx + dy * dy) + dz * dz
        dist = jnp.minimum(dist, d)
        m = jnp.max(dist, axis=1, keepdims=True)
        far = jnp.min(jnp.where(dist == m, iota, N), axis=1, keepdims=True)
        return dist, far, acc

    _, _, acc = lax.fori_loop(
        0, npoint, step,
        (jnp.full((B, N), 1e10, F32), jnp.zeros((B, 1), I32),
         jnp.minimum(siota + biota, 0)))
    idx_ref[...] = acc


def _fps(xt, npoint):
    B3, N = xt.shape
    B = B3 // 3
    return pl.pallas_call(
        functools.partial(_fps_body, npoint=npoint, B=B, N=N),
        out_shape=(jax.ShapeDtypeStruct((B, npoint), I32),
                   jax.ShapeDtypeStruct((B, N), F32)),
    )(xt)


def _sa_mlp_body(xg_ref, c_ref, *wb_refs, T, K, nl):
    out_ref = wb_refs[-1]
    wb = wb_refs[:-1]
    xg = xg_ref[...]
    c = c_ref[...]
    cb = jnp.broadcast_to(c[:, None, :], (T, K, 3)).reshape(T * K, 3)
    h = jnp.concatenate([xg, xg - cb], axis=1)
    for i in range(nl):
        h = _apply_layer(h, wb[4 * i][...], wb[4 * i + 1][...],
                         wb[4 * i + 2][...], wb[4 * i + 3][...])
    cout = h.shape[-1]
    out_ref[...] = jnp.max(h.reshape(T, K, cout), axis=1)


def _sa_mlp(xg, c, wbs, K, T=64):
    BS = c.shape[0]
    cout = wbs[-1][0].shape[1]
    flat = _flatten(wbs)
    in_specs = [
        pl.BlockSpec((T * K, 3), lambda i: (i, 0)),
        pl.BlockSpec((T, 3), lambda i: (i, 0)),
    ] + [pl.BlockSpec(a.shape, lambda i: (0, 0)) for a in flat]
    return pl.pallas_call(
        functools.partial(_sa_mlp_body, T=T, K=K, nl=len(wbs)),
        grid=(BS // T,),
        in_specs=in_specs,
        out_specs=pl.BlockSpec((T, cout), lambda i: (i, 0)),
        out_shape=jax.ShapeDtypeStruct((BS, cout), F32),
    )(xg, c, *flat)


def _sa2_mlp_body(rows_ref, c_ref, *wb_refs, T, K, nl, cf):
    out_ref = wb_refs[-1]
    wb = wb_refs[:-1]
    rows = rows_ref[...]
    c = c_ref[...]
    cb = jnp.broadcast_to(c[:, None, :], (T, K, 3)).reshape(T * K, 3)
    h = jnp.concatenate([rows[:, :cf], rows[:, cf:] - cb], axis=1)
    for i in range(nl):
        h = _apply_layer(h, wb[4 * i][...], wb[4 * i + 1][...],
                         wb[4 * i + 2][...], wb[4 * i + 3][...])
    cout = h.shape[-1]
    out_ref[...] = jnp.max(h.reshape(T, K, cout), axis=1)


def _sa2_mlp(rows, c, wbs, K, T=32):
    BS = c.shape[0]
    cin = rows.shape[1]
    cf = cin - 3
    cout = wbs[-1][0].shape[1]
    flat = _flatten(wbs)
    in_specs = [
        pl.BlockSpec((T * K, cin), lambda i: (i, 0)),
        pl.BlockSpec((T, 3), lambda i: (i, 0)),
    ] + [pl.BlockSpec(a.shape, lambda i: (0, 0)) for a in flat]
    return pl.pallas_call(
        functools.partial(_sa2_mlp_body, T=T, K=K, nl=len(wbs), cf=cf),
        grid=(BS // T,),
        in_specs=in_specs,
        out_specs=pl.BlockSpec((T, cout), lambda i: (i, 0)),
        out_shape=jax.ShapeDtypeStruct((BS, cout), F32),
    )(rows, c, *flat)


def _dense_body(x_ref, *wb_refs, pool, raw_last, nl):
    out_ref = wb_refs[-1]
    wb = wb_refs[:-1]
    h = x_ref[...]
    for i in range(nl):
        w, b, g, beta = (wb[4 * i][...], wb[4 * i + 1][...],
                         wb[4 * i + 2][...], wb[4 * i + 3][...])
        if raw_last and i == nl - 1:
            h = jnp.dot(h, w, preferred_element_type=F32) + b
        else:
            h = _apply_layer(h, w, b, g, beta)
    if pool > 1:
        r, cout = h.shape
        h = jnp.max(h.reshape(r // pool, pool, cout), axis=1)
        out_ref[...] = h.reshape(out_ref.shape)
    else:
        out_ref[...] = h


def _dense(x, wbs, T, pool=1, raw_last=False):
    R, cin = x.shape
    cout = wbs[-1][0].shape[1]
    flat = _flatten(wbs)
    in_specs = [pl.BlockSpec((T, cin), lambda i: (i, 0))] + [
        pl.BlockSpec(a.shape, lambda i: (0, 0)) for a in flat]
    if pool > 1:
        out_specs = pl.BlockSpec((1, T // pool, cout), lambda i: (i, 0, 0))
        out_shape = jax.ShapeDtypeStruct((R // T, T // pool, cout), F32)
    else:
        out_specs = pl.BlockSpec((T, cout), lambda i: (i, 0))
        out_shape = jax.ShapeDtypeStruct((R, cout), F32)
    res = pl.pallas_call(
        functools.partial(_dense_body, pool=pool, raw_last=raw_last,
                          nl=len(wbs)),
        grid=(R // T,),
        in_specs=in_specs,
        out_specs=out_specs,
        out_shape=out_shape,
    )(x, *flat)
    return res.reshape(R // pool, cout)


def _top3_weights(x1, x2t, R, S):
    s1 = jnp.sum(x1 * x1, axis=1, keepdims=True)
    s2 = jnp.sum(x2t * x2t, axis=0, keepdims=True)
    dot = jnp.dot(x1, x2t, preferred_element_type=F32)
    d = (-2.0 * dot + s1) + s2
    iota = lax.broadcasted_iota(I32, (R, S), 1)
    recips = []
    onehots = []
    for _ in range(3):
        mn = jnp.min(d, axis=1, keepdims=True)
        idx = jnp.min(jnp.where(d == mn, iota, S), axis=1, keepdims=True)
        oh = iota == idx
        recips.append(1.0 / (mn + 1e-8))
        onehots.append(oh)
        d = jnp.where(oh, jnp.inf, d)
    rsum = (recips[0] + recips[1]) + recips[2]
    wmat = jnp.zeros((R, S), F32)
    for oh, rc in zip(onehots, recips):
        wmat = wmat + jnp.where(oh, rc / rsum, 0.0)
    return wmat


def _fp_body(x1_ref, x2t_ref, p1_ref, p2_ref, *wb_refs, S, R, nl):
    out_ref = wb_refs[-1]
    wb = wb_refs[:-1]
    wmat = _top3_weights(x1_ref[0], x2t_ref[0], R, S)
    interp = jnp.dot(wmat, p2_ref[0], preferred_element_type=F32,
                     precision=HI)
    h = jnp.concatenate([p1_ref[0], interp], axis=1)
    for i in range(nl):
        h = _apply_layer(h, wb[4 * i][...], wb[4 * i + 1][...],
                         wb[4 * i + 2][...], wb[4 * i + 3][...])
    out_ref[0] = h


def _fp(x1, x2t, p1, p2, wbs, R):
    B, N, _ = x1.shape
    S = x2t.shape[2]
    cout = wbs[-1][0].shape[1]
    c1 = p1.shape[2]
    c2 = p2.shape[2]
    flat = _flatten(wbs)
    in_specs = [
        pl.BlockSpec((1, R, 3), lambda b, i: (b, i, 0)),
        pl.BlockSpec((1, 3, S), lambda b, i: (b, 0, 0)),
        pl.BlockSpec((1, R, c1), lambda b, i: (b, i, 0)),
        pl.BlockSpec((1, S, c2), lambda b, i: (b, 0, 0)),
    ] + [pl.BlockSpec(a.shape, lambda b, i: (0, 0)) for a in flat]
    return pl.pallas_call(
        functools.partial(_fp_body, S=S, R=R, nl=len(wbs)),
        grid=(B, N // R),
        in_specs=in_specs,
        out_specs=pl.BlockSpec((1, R, cout), lambda b, i: (b, i, 0)),
        out_shape=jax.ShapeDtypeStruct((B, N, cout), F32),
    )(x1, x2t, p1, p2, *flat)


def _fp1_head_body(x1_ref, x2t_ref, p2_ref, wa_ref, wbx_ref, w1i_ref,
                   b1_ref, g1_ref, be1_ref, *wb_refs, S, R, nl):
    out_ref = wb_refs[-1]
    wb = wb_refs[:-1]
    x1 = x1_ref[0]
    wmat = _top3_weights(x1, x2t_ref[0], R, S)
    interp = jnp.dot(wmat, p2_ref[0], preferred_element_type=F32,
                     precision=HI)
    h = (jnp.dot(x1, wa_ref[...], preferred_element_type=F32)
         + jnp.dot(x1, wbx_ref[...], preferred_element_type=F32)
         + jnp.dot(interp, w1i_ref[...], preferred_element_type=F32)
         + b1_ref[...])
    h = jnp.maximum(h * g1_ref[...] / SQ + be1_ref[...], 0.0)
    for i in range(nl):
        w, b, g, beta = (wb[4 * i][...], wb[4 * i + 1][...],
                         wb[4 * i + 2][...], wb[4 * i + 3][...])
        if i == nl - 1:
            h = jnp.dot(h, w, preferred_element_type=F32) + b
        else:
            h = _apply_layer(h, w, b, g, beta)
    out_ref[0] = h


def _fp1_head(x1, x2t, p2, wa, wbx, w1i, b1, g1, be1, wbs, R):
    B, N, _ = x1.shape
    S = x2t.shape[2]
    c2 = p2.shape[2]
    cout = wbs[-1][0].shape[1]
    c1 = w1i.shape[1]
    flat = _flatten(wbs)
    in_specs = [
        pl.BlockSpec((1, R, 3), lambda b, i: (b, i, 0)),
        pl.BlockSpec((1, 3, S), lambda b, i: (b, 0, 0)),
        pl.BlockSpec((1, S, c2), lambda b, i: (b, 0, 0)),
        pl.BlockSpec(wa.shape, lambda b, i: (0, 0)),
        pl.BlockSpec(wbx.shape, lambda b, i: (0, 0)),
        pl.BlockSpec(w1i.shape, lambda b, i: (0, 0)),
        pl.BlockSpec((1, c1), lambda b, i: (0, 0)),
        pl.BlockSpec((1, c1), lambda b, i: (0, 0)),
        pl.BlockSpec((1, c1), lambda b, i: (0, 0)),
    ] + [pl.BlockSpec(a.shape, lambda b, i: (0, 0)) for a in flat]
    return pl.pallas_call(
        functools.partial(_fp1_head_body, S=S, R=R, nl=len(wbs)),
        grid=(B, N // R),
        in_specs=in_specs,
        out_specs=pl.BlockSpec((1, R, cout), lambda b, i: (b, i, 0)),
        out_shape=jax.ShapeDtypeStruct((B, N, cout), F32),
    )(x1, x2t, p2, wa, wbx, w1i, b1.reshape(1, -1), g1.reshape(1, -1),
      be1.reshape(1, -1), *flat)


def _ball_query(radius, nsample, xyz, new_xyz):
    B, N, _ = xyz.shape
    S = new_xyz.shape[1]
    dd = -2.0 * jnp.einsum('bnc,bmc->bnm', new_xyz, xyz)
    dd = dd + jnp.sum(new_xyz ** 2, -1)[:, :, None]
    dd = dd + jnp.sum(xyz ** 2, -1)[:, None, :]
    gi = jnp.broadcast_to(jnp.arange(N, dtype=I32), (B, S, N))
    gi = jnp.where(dd > radius ** 2, N, gi)
    gi = jnp.sort(gi, axis=-1)[:, :, :nsample]
    first = gi[:, :, :1]
    return jnp.where(gi == N, jnp.broadcast_to(first, gi.shape), gi)


def _gather(points, idx):
    return jax.vmap(lambda p, i: p[i])(points, idx)


def kernel(xyz, params):
    B, N, _ = xyz.shape
    xt = jnp.transpose(xyz, (2, 0, 1)).reshape(3 * B, N)

    S1 = 512
    fps1, _p2_l0 = _fps(xt, S1)
    new_xyz1 = _gather(xyz, fps1)
    c1flat = new_xyz1.reshape(B * S1, 3)
    sa1_outs = []
    for radius, K, layers in zip([0.1, 0.2, 0.4], [32, 64, 128], params['sa1']):
        gidx = _ball_query(radius, K, xyz, new_xyz1)
        gx = _gather(xyz, gidx.reshape(B, S1 * K)).reshape(B * S1 * K, 3)
        sa1_outs.append(_sa_mlp(gx, c1flat, _wbs(layers), K))
    l1_points = jnp.concatenate(sa1_outs, axis=1)

    S2 = 128
    x1t = jnp.transpose(new_xyz1, (2, 0, 1)).reshape(3 * B, S1)
    fps2, _p2_l1 = _fps(x1t, S2)
    new_xyz2 = _gather(new_xyz1, fps2)
    c2flat = new_xyz2.reshape(B * S2, 3)
    table = jnp.concatenate([l1_points, c1flat], axis=1)
    sa2_outs = []
    for radius, K, layers in zip([0.4, 0.8], [64, 128], params['sa2']):
        gidx = _ball_query(radius, K, new_xyz1, new_xyz2)
        base = (jnp.arange(B, dtype=I32) * S1)[:, None, None]
        rows = (gidx + base).reshape(B * S2 * K)
        sa2_outs.append(_sa2_mlp(table[rows], c2flat, _wbs(layers), K))
    l2_points = jnp.concatenate(sa2_outs, axis=1)

    x3 = jnp.concatenate([c2flat, l2_points], axis=1)
    l3_points = _dense(x3, _wbs(params['sa3']), T=S2, pool=S2)

    l2p = l2_points.reshape(B, S2, 512)
    x_fp3 = jnp.concatenate(
        [l2p, jnp.broadcast_to(l3_points[:, None, :], (B, S2, 1024))], axis=2)
    l2p = _dense(x_fp3.reshape(B * S2, 1536), _wbs(params['fp3']),
                 T=S2)

    x2t_b = jnp.transpose(new_xyz2, (0, 2, 1))
    l1p = _fp(new_xyz1, x2t_b, l1_points.reshape(B, S1, 320),
              l2p.reshape(B, S2, 256), _wbs(params['fp2']), R=S1)

    x1t_b = jnp.transpose(new_xyz1, (0, 2, 1))
    w1, b1, g1, be1 = _wbs(params['fp1'])[0]
    wa = w1[16:19]
    wbx = w1[19:22]
    w1i = w1[22:]
    zb = jnp.zeros((1, 50), F32)
    wbs_tail = _wbs(params['fp1'])[1:] + _wbs(params['head_conv1'])
    wbs_tail = wbs_tail + [(params['head_W2'],
                            params['head_b2'].reshape(1, -1), zb, zb)]
    out = _fp1_head(xyz, x1t_b, l1p, wa, wbx, w1i, b1, g1, be1, wbs_tail,
                    R=1024)
    return out


# speedup vs baseline: 16.3821x; 10.0007x over previous
import functools

import jax
import jax.numpy as jnp
import numpy as np
from jax import lax
from jax.experimental import pallas as pl
from jax.experimental.pallas import tpu as pltpu
from jax.experimental.pallas import tpu_sc as plsc

BN_EPS = 1e-5
SQ = float(np.sqrt(np.float32(1.0 + BN_EPS)))
F32 = jnp.float32
I32 = jnp.int32
HI = jax.lax.Precision.HIGHEST


def _wbs(layers):
    out = []
    for p in layers:
        out.append((p['W'], p['b'].reshape(1, -1), p['gamma'].reshape(1, -1),
                    p['beta'].reshape(1, -1)))
    return out


def _apply_layer(h, w, b, g, beta):
    h = jnp.dot(h, w, preferred_element_type=F32) + b
    h = h * g / SQ + beta
    return jnp.maximum(h, 0.0)


def _flatten(wbs):
    flat = []
    for t in wbs:
        flat.extend(t)
    return flat


def _fps_body(xt_ref, idx_ref, p2_ref, *, npoint, B, N):
    x = xt_ref[0:B, :]
    y = xt_ref[B:2 * B, :]
    z = xt_ref[2 * B:3 * B, :]
    p2_ref[...] = (x * x + y * y) + z * z
    iota = lax.broadcasted_iota(I32, (B, N), 1)
    siota = lax.broadcasted_iota(I32, (B, npoint), 1)
    biota = lax.broadcasted_iota(I32, (B, npoint), 0)

    def step(t, carry):
        dist, far, acc = carry
        acc = jnp.where(siota == t, far, acc)
        oh = iota == far
        cx = jnp.sum(jnp.where(oh, x, 0.0), axis=1, keepdims=True)
        cy = jnp.sum(jnp.where(oh, y, 0.0), axis=1, keepdims=True)
        cz = jnp.sum(jnp.where(oh, z, 0.0), axis=1, keepdims=True)
        dx = x - cx
        dy = y - cy
        dz = z - cz
        d = (dx * dx + dy * dy) + dz * dz
        dist = jnp.minimum(dist, d)
        m = jnp.max(dist, axis=1, keepdims=True)
        far = jnp.min(jnp.where(dist == m, iota, N), axis=1, keepdims=True)
        return dist, far, acc

    _, _, acc = lax.fori_loop(
        0, npoint, step,
        (jnp.full((B, N), 1e10, F32), jnp.zeros((B, 1), I32),
         jnp.minimum(siota + biota, 0)))
    idx_ref[...] = acc


def _fps(xt, npoint):
    B3, N = xt.shape
    B = B3 // 3
    return pl.pallas_call(
        functools.partial(_fps_body, npoint=npoint, B=B, N=N),
        out_shape=(jax.ShapeDtypeStruct((B, npoint), I32),
                   jax.ShapeDtypeStruct((B, N), F32)),
    )(xt)


def _sa_mlp_body(xg_ref, c_ref, *wb_refs, T, K, nl):
    out_ref = wb_refs[-1]
    wb = wb_refs[:-1]
    xg = xg_ref[...]
    c = c_ref[...]
    cb = jnp.broadcast_to(c[:, None, :], (T, K, 3)).reshape(T * K, 3)
    h = jnp.concatenate([xg, xg - cb], axis=1)
    for i in range(nl):
        h = _apply_layer(h, wb[4 * i][...], wb[4 * i + 1][...],
                         wb[4 * i + 2][...], wb[4 * i + 3][...])
    cout = h.shape[-1]
    out_ref[...] = jnp.max(h.reshape(T, K, cout), axis=1)


def _sa_mlp(xg, c, wbs, K, T=64):
    BS = c.shape[0]
    cout = wbs[-1][0].shape[1]
    flat = _flatten(wbs)
    in_specs = [
        pl.BlockSpec((T * K, 3), lambda i: (i, 0)),
        pl.BlockSpec((T, 3), lambda i: (i, 0)),
    ] + [pl.BlockSpec(a.shape, lambda i: (0, 0)) for a in flat]
    return pl.pallas_call(
        functools.partial(_sa_mlp_body, T=T, K=K, nl=len(wbs)),
        grid=(BS // T,),
        in_specs=in_specs,
        out_specs=pl.BlockSpec((T, cout), lambda i: (i, 0)),
        out_shape=jax.ShapeDtypeStruct((BS, cout), F32),
    )(xg, c, *flat)


def _sa2_mlp_body(rows_ref, c_ref, *wb_refs, T, K, nl, cf):
    out_ref = wb_refs[-1]
    wb = wb_refs[:-1]
    rows = rows_ref[...]
    c = c_ref[...]
    cb = jnp.broadcast_to(c[:, None, :], (T, K, 3)).reshape(T * K, 3)
    h = jnp.concatenate([rows[:, :cf], rows[:, cf:cf + 3] - cb], axis=1)
    for i in range(nl):
        h = _apply_layer(h, wb[4 * i][...], wb[4 * i + 1][...],
                         wb[4 * i + 2][...], wb[4 * i + 3][...])
    cout = h.shape[-1]
    out_ref[...] = jnp.max(h.reshape(T, K, cout), axis=1)


def _sa2_mlp(rows, c, wbs, K, T=32, cf=320):
    BS = c.shape[0]
    cin = rows.shape[1]
    cout = wbs[-1][0].shape[1]
    flat = _flatten(wbs)
    in_specs = [
        pl.BlockSpec((T * K, cin), lambda i: (i, 0)),
        pl.BlockSpec((T, 3), lambda i: (i, 0)),
    ] + [pl.BlockSpec(a.shape, lambda i: (0, 0)) for a in flat]
    return pl.pallas_call(
        functools.partial(_sa2_mlp_body, T=T, K=K, nl=len(wbs), cf=cf),
        grid=(BS // T,),
        in_specs=in_specs,
        out_specs=pl.BlockSpec((T, cout), lambda i: (i, 0)),
        out_shape=jax.ShapeDtypeStruct((BS, cout), F32),
    )(rows, c, *flat)


def _dense_body(x_ref, *wb_refs, pool, raw_last, nl):
    out_ref = wb_refs[-1]
    wb = wb_refs[:-1]
    h = x_ref[...]
    for i in range(nl):
        w, b, g, beta = (wb[4 * i][...], wb[4 * i + 1][...],
                         wb[4 * i + 2][...], wb[4 * i + 3][...])
        if raw_last and i == nl - 1:
            h = jnp.dot(h, w, preferred_element_type=F32) + b
        else:
            h = _apply_layer(h, w, b, g, beta)
    if pool > 1:
        r, cout = h.shape
        h = jnp.max(h.reshape(r // pool, pool, cout), axis=1)
        out_ref[...] = h.reshape(out_ref.shape)
    else:
        out_ref[...] = h


def _dense(x, wbs, T, pool=1, raw_last=False):
    R, cin = x.shape
    cout = wbs[-1][0].shape[1]
    flat = _flatten(wbs)
    in_specs = [pl.BlockSpec((T, cin), lambda i: (i, 0))] + [
        pl.BlockSpec(a.shape, lambda i: (0, 0)) for a in flat]
    if pool > 1:
        out_specs = pl.BlockSpec((1, T // pool, cout), lambda i: (i, 0, 0))
        out_shape = jax.ShapeDtypeStruct((R // T, T // pool, cout), F32)
    else:
        out_specs = pl.BlockSpec((T, cout), lambda i: (i, 0))
        out_shape = jax.ShapeDtypeStruct((R, cout), F32)
    res = pl.pallas_call(
        functools.partial(_dense_body, pool=pool, raw_last=raw_last,
                          nl=len(wbs)),
        grid=(R // T,),
        in_specs=in_specs,
        out_specs=out_specs,
        out_shape=out_shape,
    )(x, *flat)
    return res.reshape(R // pool, cout)


def _top3_weights(x1, x2t, R, S):
    s1 = jnp.sum(x1 * x1, axis=1, keepdims=True)
    s2 = jnp.sum(x2t * x2t, axis=0, keepdims=True)
    dot = jnp.dot(x1, x2t, preferred_element_type=F32)
    d = (-2.0 * dot + s1) + s2
    iota = lax.broadcasted_iota(I32, (R, S), 1)
    recips = []
    onehots = []
    for _ in range(3):
        mn = jnp.min(d, axis=1, keepdims=True)
        idx = jnp.min(jnp.where(d == mn, iota, S), axis=1, keepdims=True)
        oh = iota == idx
        recips.append(1.0 / (mn + 1e-8))
        onehots.append(oh)
        d = jnp.where(oh, jnp.inf, d)
    rsum = (recips[0] + recips[1]) + recips[2]
    wmat = jnp.zeros((R, S), F32)
    for oh, rc in zip(onehots, recips):
        wmat = wmat + jnp.where(oh, rc / rsum, 0.0)
    return wmat


def _fp_body(x1_ref, x2t_ref, p1_ref, p2_ref, *wb_refs, S, R, nl):
    out_ref = wb_refs[-1]
    wb = wb_refs[:-1]
    wmat = _top3_weights(x1_ref[0], x2t_ref[0], R, S)
    interp = jnp.dot(wmat, p2_ref[0], preferred_element_type=F32,
                     precision=HI)
    h = jnp.concatenate([p1_ref[0], interp], axis=1)
    for i in range(nl):
        h = _apply_layer(h, wb[4 * i][...], wb[4 * i + 1][...],
                         wb[4 * i + 2][...], wb[4 * i + 3][...])
    out_ref[0] = h


def _fp(x1, x2t, p1, p2, wbs, R):
    B, N, _ = x1.shape
    S = x2t.shape[2]
    cout = wbs[-1][0].shape[1]
    c1 = p1.shape[2]
    c2 = p2.shape[2]
    flat = _flatten(wbs)
    in_specs = [
        pl.BlockSpec((1, R, 3), lambda b, i: (b, i, 0)),
        pl.BlockSpec((1, 3, S), lambda b, i: (b, 0, 0)),
        pl.BlockSpec((1, R, c1), lambda b, i: (b, i, 0)),
        pl.BlockSpec((1, S, c2), lambda b, i: (b, 0, 0)),
    ] + [pl.BlockSpec(a.shape, lambda b, i: (0, 0)) for a in flat]
    return pl.pallas_call(
        functools.partial(_fp_body, S=S, R=R, nl=len(wbs)),
        grid=(B, N // R),
        in_specs=in_specs,
        out_specs=pl.BlockSpec((1, R, cout), lambda b, i: (b, i, 0)),
        out_shape=jax.ShapeDtypeStruct((B, N, cout), F32),
    )(x1, x2t, p1, p2, *flat)


def _fp1_head_body(x1_ref, x2t_ref, p2_ref, wa_ref, wbx_ref, w1i_ref,
                   b1_ref, g1_ref, be1_ref, *wb_refs, S, R, nl):
    out_ref = wb_refs[-1]
    wb = wb_refs[:-1]
    x1 = x1_ref[0]
    wmat = _top3_weights(x1, x2t_ref[0], R, S)
    interp = jnp.dot(wmat, p2_ref[0], preferred_element_type=F32,
                     precision=HI)
    h = (jnp.dot(x1, wa_ref[...], preferred_element_type=F32)
         + jnp.dot(x1, wbx_ref[...], preferred_element_type=F32)
         + jnp.dot(interp, w1i_ref[...], preferred_element_type=F32)
         + b1_ref[...])
    h = jnp.maximum(h * g1_ref[...] / SQ + be1_ref[...], 0.0)
    for i in range(nl):
        w, b, g, beta = (wb[4 * i][...], wb[4 * i + 1][...],
                         wb[4 * i + 2][...], wb[4 * i + 3][...])
        if i == nl - 1:
            h = jnp.dot(h, w, preferred_element_type=F32) + b
        else:
            h = _apply_layer(h, w, b, g, beta)
    out_ref[0] = h


def _fp1_head(x1, x2t, p2, wa, wbx, w1i, b1, g1, be1, wbs, R):
    B, N, _ = x1.shape
    S = x2t.shape[2]
    c2 = p2.shape[2]
    cout = wbs[-1][0].shape[1]
    c1 = w1i.shape[1]
    flat = _flatten(wbs)
    in_specs = [
        pl.BlockSpec((1, R, 3), lambda b, i: (b, i, 0)),
        pl.BlockSpec((1, 3, S), lambda b, i: (b, 0, 0)),
        pl.BlockSpec((1, S, c2), lambda b, i: (b, 0, 0)),
        pl.BlockSpec(wa.shape, lambda b, i: (0, 0)),
        pl.BlockSpec(wbx.shape, lambda b, i: (0, 0)),
        pl.BlockSpec(w1i.shape, lambda b, i: (0, 0)),
        pl.BlockSpec((1, c1), lambda b, i: (0, 0)),
        pl.BlockSpec((1, c1), lambda b, i: (0, 0)),
        pl.BlockSpec((1, c1), lambda b, i: (0, 0)),
    ] + [pl.BlockSpec(a.shape, lambda b, i: (0, 0)) for a in flat]
    return pl.pallas_call(
        functools.partial(_fp1_head_body, S=S, R=R, nl=len(wbs)),
        grid=(B, N // R),
        in_specs=in_specs,
        out_specs=pl.BlockSpec((1, R, cout), lambda b, i: (b, i, 0)),
        out_shape=jax.ShapeDtypeStruct((B, N, cout), F32),
    )(x1, x2t, p2, wa, wbx, w1i, b1.reshape(1, -1), g1.reshape(1, -1),
      be1.reshape(1, -1), *flat)


_LANES = 16


def _rbf16(v):
    bits = lax.bitcast_convert_type(v, I32)
    rnd = jnp.bitwise_and(lax.shift_right_logical(bits, 16), 1)
    bits = jnp.bitwise_and(bits + 32767 + rnd, -65536)
    return lax.bitcast_convert_type(bits, F32)


def _iota16():
    return lax.broadcasted_iota(I32, (_LANES,), 0)


def _sload(ref, i):
    return ref[pl.ds(i, _LANES)][0]


def _sc_ball1(xt, p2, fps, B, N, S, radii, ks):
    n_br = len(radii)
    r2s = [float(np.float32(r * r)) for r in radii]
    cpw = (B * S) // 32
    wpb = 32 // B
    mesh = plsc.VectorSubcoreMesh(core_axis_name="c", subcore_axis_name="s")
    kmax = max(ks)
    scratch = ([pltpu.VMEM((N,), F32)] * 3
               + [pltpu.VMEM((N,), F32)] * 3
               + [pltpu.VMEM((N,), F32)]
               + [pltpu.VMEM((cpw,), I32)]
               + [pltpu.VMEM((cpw + _LANES,), F32)] * 7
               + [pltpu.VMEM((N + _LANES,), I32)] * n_br
               + [pltpu.VMEM((kmax,), F32)] * 3)

    @functools.partial(
        pl.kernel,
        out_type=tuple([jax.ShapeDtypeStruct((B * S * 3 * k,), F32)
                        for k in ks]
                       + [jax.ShapeDtypeStruct((3 * B * S,), F32)]),
        mesh=mesh,
        scratch_types=scratch,
        compiler_params=pltpu.CompilerParams(needs_layout_passes=False),
    )
    def k(xt_h, p2_h, fps_h, *refs):
        outs = refs[:n_br]
        c_h = refs[n_br]
        xr, yr, zr, xb, yb, zb, pp = refs[n_br + 1:n_br + 8]
        ci = refs[n_br + 8]
        cxr, cyr, czr, cxb, cyb, czb, c2 = refs[n_br + 9:n_br + 16]
        ibufs = refs[n_br + 16:n_br + 16 + n_br]
        gxs, gys, gzs = refs[n_br + 16 + n_br:]
        wid = lax.axis_index("s") * 2 + lax.axis_index("c")
        b = wid // wpb
        cbase = (wid % wpb) * cpw
        pltpu.sync_copy(xt_h.at[pl.ds(b * N, N)], xr)
        pltpu.sync_copy(xt_h.at[pl.ds((B + b) * N, N)], yr)
        pltpu.sync_copy(xt_h.at[pl.ds((2 * B + b) * N, N)], zr)
        pltpu.sync_copy(p2_h.at[pl.ds(b * N, N)], pp)
        pltpu.sync_copy(fps_h.at[pl.ds(b * S + cbase, cpw)], ci)

        def round_chunk(i, _):
            sl = pl.ds(i * _LANES, _LANES)
            xb[sl] = _rbf16(xr[sl])
            yb[sl] = _rbf16(yr[sl])
            zb[sl] = _rbf16(zr[sl])
            return 0
        lax.fori_loop(0, N // _LANES, round_chunk, 0)

        for i in range(cpw // _LANES):
            sl = pl.ds(i * _LANES, _LANES)
            idx = ci[sl]
            gx = plsc.load_gather(xr, [idx])
            gy = plsc.load_gather(yr, [idx])
            gz = plsc.load_gather(zr, [idx])
            cxr[sl] = gx
            cyr[sl] = gy
            czr[sl] = gz
            cxb[sl] = _rbf16(gx)
            cyb[sl] = _rbf16(gy)
            czb[sl] = _rbf16(gz)
            c2[sl] = plsc.load_gather(pp, [idx])
        pltpu.sync_copy(cxr.at[pl.ds(0, cpw)],
                        c_h.at[pl.ds(b * S + cbase, cpw)])
        pltpu.sync_copy(cyr.at[pl.ds(0, cpw)],
                        c_h.at[pl.ds((B + b) * S + cbase, cpw)])
        pltpu.sync_copy(czr.at[pl.ds(0, cpw)],
                        c_h.at[pl.ds((2 * B + b) * S + cbase, cpw)])

        def per_center(j, _):
            cx = _sload(cxb, j)
            cy = _sload(cyb, j)
            cz = _sload(czb, j)
            c2s = _sload(c2, j)
            row = b * S + cbase + j

            def chunk(kk, cnts):
                sl = pl.ds(kk * _LANES, _LANES)
                d = (-2.0 * ((cx * xb[sl] + cy * yb[sl]) + cz * zb[sl])
                     + c2s) + pp[sl]
                jv = _iota16() + (kk * _LANES)
                new = []
                for t in range(n_br):
                    m = d <= r2s[t]
                    plsc.store_compressed(
                        ibufs[t].at[pl.ds(cnts[t], _LANES)], jv, mask=m)
                    new.append(
                        cnts[t] + plsc.all_reduce_population_count(m)[0])
                return tuple(new)

            cnts = lax.fori_loop(0, N // _LANES, chunk,
                                 tuple(jnp.int32(0) for _ in range(n_br)))
            for t in range(n_br):
                kt = ks[t]
                cc = jnp.minimum(cnts[t], kt)
                pad = lax.cond(cnts[t] > 0,
                               lambda: _sload(ibufs[t], 0),
                               lambda: jnp.int32(N - 1))
                for i in range(kt // _LANES):
                    sl = pl.ds(i * _LANES, _LANES)
                    lane = _iota16() + (i * _LANES)
                    idx = jnp.where(lane >= cc, pad, ibufs[t][sl])
                    gxs[sl] = plsc.load_gather(xr, [idx])
                    gys[sl] = plsc.load_gather(yr, [idx])
                    gzs[sl] = plsc.load_gather(zr, [idx])
                pltpu.sync_copy(gxs.at[pl.ds(0, kt)],
                                outs[t].at[pl.ds((row * 3) * kt, kt)])
                pltpu.sync_copy(gys.at[pl.ds(0, kt)],
                                outs[t].at[pl.ds((row * 3 + 1) * kt, kt)])
                pltpu.sync_copy(gzs.at[pl.ds(0, kt)],
                                outs[t].at[pl.ds((row * 3 + 2) * kt, kt)])
            return 0

        lax.fori_loop(0, cpw, per_center, 0)

    return k(xt, p2, fps)


def _sc_ball2(xt, p2, fps, table, B, N, S, radii, ks):
    n_br = len(radii)
    D = table.shape[1]
    r2s = [float(np.float32(r * r)) for r in radii]
    cpw = (B * S) // 32
    wpb = 32 // B
    mesh = plsc.VectorSubcoreMesh(core_axis_name="c", subcore_axis_name="s")
    scratch = ([pltpu.VMEM((N,), F32)] * 3
               + [pltpu.VMEM((N,), F32)]
               + [pltpu.VMEM((cpw,), I32)]
               + [pltpu.VMEM((cpw + _LANES,), F32)] * 7
               + [pltpu.VMEM((N + _LANES,), I32)] * n_br
               + [pltpu.VMEM((k,), I32) for k in ks]
               + [pltpu.VMEM((k, D), F32) for k in ks]
               + [pltpu.SemaphoreType.DMA])

    @functools.partial(
        pl.kernel,
        out_type=tuple([jax.ShapeDtypeStruct((B * S * k, D), F32) for k in ks]
                       + [jax.ShapeDtypeStruct((3 * B * S,), F32)]),
        mesh=mesh,
        scratch_types=scratch,
        compiler_params=pltpu.CompilerParams(needs_layout_passes=False),
    )
    def k(xt_h, p2_h, fps_h, tab_h, *refs):
        outs = refs[:n_br]
        c_h = refs[n_br]
        xb, yb, zb, pp = refs[n_br + 1:n_br + 5]
        ci = refs[n_br + 5]
        cxr, cyr, czr, cxb, cyb, czb, c2 = refs[n_br + 6:n_br + 13]
        bufs = refs[n_br + 13:n_br + 13 + n_br]
        idxs = refs[n_br + 13 + n_br:n_br + 13 + 2 * n_br]
        rows = refs[n_br + 13 + 2 * n_br:n_br + 13 + 3 * n_br]
        sem = refs[-1]
        wid = lax.axis_index("s") * 2 + lax.axis_index("c")
        b = wid // wpb
        cbase = (wid % wpb) * cpw
        pltpu.sync_copy(xt_h.at[pl.ds(b * N, N)], xb)
        pltpu.sync_copy(xt_h.at[pl.ds((B + b) * N, N)], yb)
        pltpu.sync_copy(xt_h.at[pl.ds((2 * B + b) * N, N)], zb)
        pltpu.sync_copy(p2_h.at[pl.ds(b * N, N)], pp)
        pltpu.sync_copy(fps_h.at[pl.ds(b * S + cbase, cpw)], ci)

        for i in range(cpw // _LANES):
            sl = pl.ds(i * _LANES, _LANES)
            idx = ci[sl]
            gx = plsc.load_gather(xb, [idx])
            gy = plsc.load_gather(yb, [idx])
            gz = plsc.load_gather(zb, [idx])
            cxr[sl] = gx
            cyr[sl] = gy
            czr[sl] = gz
            cxb[sl] = _rbf16(gx)
            cyb[sl] = _rbf16(gy)
            czb[sl] = _rbf16(gz)
            c2[sl] = plsc.load_gather(pp, [idx])
        pltpu.sync_copy(cxr.at[pl.ds(0, cpw)],
                        c_h.at[pl.ds(b * S + cbase, cpw)])
        pltpu.sync_copy(cyr.at[pl.ds(0, cpw)],
                        c_h.at[pl.ds((B + b) * S + cbase, cpw)])
        pltpu.sync_copy(czr.at[pl.ds(0, cpw)],
                        c_h.at[pl.ds((2 * B + b) * S + cbase, cpw)])

        def round_chunk(i, _):
            sl = pl.ds(i * _LANES, _LANES)
            xb[sl] = _rbf16(xb[sl])
            yb[sl] = _rbf16(yb[sl])
            zb[sl] = _rbf16(zb[sl])
            return 0
        lax.fori_loop(0, N // _LANES, round_chunk, 0)

        def per_center(j, _):
            cx = _sload(cxb, j)
            cy = _sload(cyb, j)
            cz = _sload(czb, j)
            c2s = _sload(c2, j)
            row = b * S + cbase + j

            def chunk(kk, cnts):
                sl = pl.ds(kk * _LANES, _LANES)
                d = (-2.0 * ((cx * xb[sl] + cy * yb[sl]) + cz * zb[sl])
                     + c2s) + pp[sl]
                jv = (_iota16() + (kk * _LANES)) + b * N
                new = []
                for t in range(n_br):
                    m = d <= r2s[t]
                    plsc.store_compressed(
                        bufs[t].at[pl.ds(cnts[t], _LANES)], jv, mask=m)
                    new.append(
                        cnts[t] + plsc.all_reduce_population_count(m)[0])
                return tuple(new)

            cnts = lax.fori_loop(0, N // _LANES, chunk,
                                 tuple(jnp.int32(0) for _ in range(n_br)))
            for t in range(n_br):
                kt = ks[t]
                cc = jnp.minimum(cnts[t], kt)
                pad = lax.cond(cnts[t] > 0,
                               lambda: _sload(bufs[t], 0),
                               lambda: b * N + (N - 1))
                for i in range(kt // _LANES):
                    sl = pl.ds(i * _LANES, _LANES)
                    lane = _iota16() + (i * _LANES)
                    idxs[t][sl] = jnp.where(lane >= cc, pad, bufs[t][sl])
                pltpu.async_copy(tab_h.at[idxs[t]], rows[t], sem).wait()
                pltpu.sync_copy(rows[t], outs[t].at[pl.ds(row * kt, kt)])
            return 0

        lax.fori_loop(0, cpw, per_center, 0)

    return k(xt, p2, fps, table)


def kernel(xyz, params):
    B, N, _ = xyz.shape
    xt = jnp.transpose(xyz, (2, 0, 1)).reshape(3 * B, N)

    S1 = 512
    fps1, p2_l0 = _fps(xt, S1)
    v0, v1, v2, c1t = _sc_ball1(xt.reshape(-1), p2_l0.reshape(-1),
                                fps1.reshape(-1), B, N, S1,
                                [0.1, 0.2, 0.4], [32, 64, 128])
    c1t = c1t.reshape(3, B, S1)
    new_xyz1 = jnp.transpose(c1t, (1, 2, 0))
    c1flat = new_xyz1.reshape(B * S1, 3)
    sa1_outs = []
    for vb, K, layers in zip([v0, v1, v2], [32, 64, 128], params['sa1']):
        gx = jnp.transpose(vb.reshape(B * S1, 3, K),
                           (0, 2, 1)).reshape(B * S1 * K, 3)
        sa1_outs.append(_sa_mlp(gx, c1flat, _wbs(layers), K))
    l1_points = jnp.concatenate(sa1_outs, axis=1)

    S2 = 128
    x1t = c1t.reshape(3 * B, S1)
    fps2, p2_l1 = _fps(x1t, S2)
    table = jnp.concatenate(
        [l1_points, c1flat, jnp.zeros((B * S1, 61), F32)], axis=1)
    g0, g1, c2t = _sc_ball2(x1t.reshape(-1), p2_l1.reshape(-1),
                            fps2.reshape(-1), table, B, S1, S2,
                            [0.4, 0.8], [64, 128])
    c2t = c2t.reshape(3, B, S2)
    new_xyz2 = jnp.transpose(c2t, (1, 2, 0))
    c2flat = new_xyz2.reshape(B * S2, 3)
    sa2_outs = []
    for g, K, layers in zip([g0, g1], [64, 128], params['sa2']):
        sa2_outs.append(_sa2_mlp(g, c2flat, _wbs(layers), K))
    l2_points = jnp.concatenate(sa2_outs, axis=1)

    x3 = jnp.concatenate([c2flat, l2_points], axis=1)
    l3_points = _dense(x3, _wbs(params['sa3']), T=S2, pool=S2)

    l2p = l2_points.reshape(B, S2, 512)
    x_fp3 = jnp.concatenate(
        [l2p, jnp.broadcast_to(l3_points[:, None, :], (B, S2, 1024))], axis=2)
    l2p = _dense(x_fp3.reshape(B * S2, 1536), _wbs(params['fp3']),
                 T=S2)

    x2t_b = jnp.transpose(new_xyz2, (0, 2, 1))
    l1p = _fp(new_xyz1, x2t_b, l1_points.reshape(B, S1, 320),
              l2p.reshape(B, S2, 256), _wbs(params['fp2']), R=S1)

    x1t_b = jnp.transpose(new_xyz1, (0, 2, 1))
    w1, b1, g1, be1 = _wbs(params['fp1'])[0]
    wa = w1[16:19]
    wbx = w1[19:22]
    w1i = w1[22:]
    zb = jnp.zeros((1, 50), F32)
    wbs_tail = _wbs(params['fp1'])[1:] + _wbs(params['head_conv1'])
    wbs_tail = wbs_tail + [(params['head_W2'],
                            params['head_b2'].reshape(1, -1), zb, zb)]
    out = _fp1_head(xyz, x1t_b, l1p, wa, wbx, w1i, b1, g1, be1, wbs_tail,
                    R=1024)
    return out
